# baseline (speedup 1.0000x reference)
import functools

import jax
import jax.numpy as jnp
from jax import lax
from jax.experimental import pallas as pl
from jax.experimental.pallas import tpu as pltpu
from jax.experimental.pallas import tpu_sc as plsc

B = 4096
D = 32
L = 16
NC = 2
NS = 16
NW = NC * NS
BPW = B // NW
PACK = 128 // D
VBIG = 1000000 // PACK

_mesh = plsc.VectorSubcoreMesh(core_axis_name="c", subcore_axis_name="s")


@functools.partial(
    pl.kernel,
    mesh=_mesh,
    out_type=jax.ShapeDtypeStruct((B,), jnp.float32),
    scratch_types=[
        pltpu.VMEM((BPW,), jnp.int32),
        pltpu.VMEM((BPW,), jnp.int32),
        pltpu.VMEM((BPW,), jnp.int32),
        pltpu.VMEM((BPW,), jnp.int32),
        pltpu.VMEM((BPW, 128), jnp.float32),
        pltpu.VMEM((BPW, 128), jnp.float32),
        pltpu.VMEM((BPW,), jnp.float32),
        pltpu.SemaphoreType.DMA,
        pltpu.SemaphoreType.DMA,
    ],
    compiler_params=pltpu.CompilerParams(needs_layout_passes=False),
)
def _neg_sampling_dot(idx_c_hbm, idx_w_hbm, ctx_hbm, word_hbm, out_hbm,
                      idx_c_v, idx_w_v, big_c_v, big_w_v, rows_c, rows_w,
                      acc, sem_c, sem_w):
    wid = lax.axis_index("s") * NC + lax.axis_index("c")
    base = wid * BPW

    pltpu.sync_copy(idx_c_hbm.at[pl.ds(base, BPW)], idx_c_v)
    pltpu.sync_copy(idx_w_hbm.at[pl.ds(base, BPW)], idx_w_v)

    for blk in range(BPW // L):
        sl = pl.ds(blk * L, L)
        big_c_v[sl] = idx_c_v[sl] >> 2
        big_w_v[sl] = idx_w_v[sl] >> 2

    cp_c = pltpu.async_copy(ctx_hbm.at[big_c_v], rows_c, sem_c)
    cp_w = pltpu.async_copy(word_hbm.at[big_w_v], rows_w, sem_w)
    cp_c.wait()
    cp_w.wait()

    lane = lax.iota(jnp.int32, L)
    for blk in range(BPW // L):
        sl = pl.ds(blk * L, L)
        rows16 = lane + blk * L
        cbase_c = (idx_c_v[sl] & (PACK - 1)) * D
        cbase_w = (idx_w_v[sl] & (PACK - 1)) * D
        acc_v = jnp.zeros((L,), jnp.float32)
        for d in range(D):
            wv = plsc.load_gather(rows_w, [rows16, cbase_w + d])
            cv = plsc.load_gather(rows_c, [rows16, cbase_c + d])
            acc_v = acc_v + wv * cv
        acc[sl] = acc_v

    pltpu.sync_copy(acc, out_hbm.at[pl.ds(base, BPW)])


def kernel(inputs, word_embeddings, context_embeddings):
    idx_c = inputs[:, 0].astype(jnp.int32)
    idx_w = inputs[:, 1].astype(jnp.int32)
    ctx_big = context_embeddings.reshape(VBIG, 128)
    word_big = word_embeddings.reshape(VBIG, 128)
    return _neg_sampling_dot(idx_c, idx_w, ctx_big, word_big)


# speedup vs baseline: 9.4989x; 9.4989x over previous
import functools

import jax
import jax.numpy as jnp
from jax import lax
from jax.experimental import pallas as pl
from jax.experimental.pallas import tpu as pltpu
from jax.experimental.pallas import tpu_sc as plsc

B = 4096
D = 32
L = 16
NC = 2
NS = 16
NW = NC * NS
BPW = B // NW
WAVE = 8

_mesh = plsc.VectorSubcoreMesh(core_axis_name="c", subcore_axis_name="s")


@functools.partial(
    pl.kernel,
    mesh=_mesh,
    out_type=jax.ShapeDtypeStruct((B,), jnp.float32),
    scratch_types=[
        pltpu.VMEM((BPW,), jnp.int32),
        pltpu.VMEM((BPW,), jnp.int32),
        pltpu.VMEM((WAVE, D, 128), jnp.float32),
        pltpu.VMEM((WAVE, D, 128), jnp.float32),
        pltpu.VMEM((BPW, D), jnp.float32),
        pltpu.VMEM((BPW, D), jnp.float32),
        pltpu.VMEM((BPW,), jnp.float32),
        pltpu.SemaphoreType.DMA,
    ],
    compiler_params=pltpu.CompilerParams(needs_layout_passes=False),
)
def _neg_sampling_dot(idx_c_hbm, idx_w_hbm, ctx_t_hbm, word_t_hbm, out_hbm,
                      idx_c_v, idx_w_v, blk_c, blk_w, rows_c, rows_w,
                      acc, sem):
    wid = lax.axis_index("s") * NC + lax.axis_index("c")
    base = wid * BPW

    pltpu.sync_copy(idx_c_hbm.at[pl.ds(base, BPW)], idx_c_v)
    pltpu.sync_copy(idx_w_hbm.at[pl.ds(base, BPW)], idx_w_v)

    sc_c, sc_w = [], []
    for blk in range(BPW // L):
        vcs = idx_c_v[pl.ds(blk * L, L)]
        vws = idx_w_v[pl.ds(blk * L, L)]
        for r in range(L):
            sc_c.append(lax.index_in_dim(vcs, r, keepdims=False))
            sc_w.append(lax.index_in_dim(vws, r, keepdims=False))

    d_lo = lax.iota(jnp.int32, L)
    d_hi = d_lo + L

    def fire_wave(w):
        cps = []
        for r in range(WAVE):
            j = w * WAVE + r
            off_c = pl.multiple_of((sc_c[j] >> 7) * 128, 128)
            off_w = pl.multiple_of((sc_w[j] >> 7) * 128, 128)
            cps.append(pltpu.async_copy(
                ctx_t_hbm.at[:, pl.ds(off_c, 128)], blk_c.at[r], sem))
            cps.append(pltpu.async_copy(
                word_t_hbm.at[:, pl.ds(off_w, 128)], blk_w.at[r], sem))
        return cps

    n_waves = BPW // WAVE
    pending = fire_wave(0)
    for w in range(n_waves):
        for cp in pending:
            cp.wait()
        for r in range(WAVE):
            j = w * WAVE + r
            vr_c = jnp.full((L,), sc_c[j] & 127, jnp.int32)
            vr_w = jnp.full((L,), sc_w[j] & 127, jnp.int32)
            rows_c[j, pl.ds(0, L)] = plsc.load_gather(blk_c.at[r], [d_lo, vr_c])
            rows_c[j, pl.ds(L, L)] = plsc.load_gather(blk_c.at[r], [d_hi, vr_c])
            rows_w[j, pl.ds(0, L)] = plsc.load_gather(blk_w.at[r], [d_lo, vr_w])
            rows_w[j, pl.ds(L, L)] = plsc.load_gather(blk_w.at[r], [d_hi, vr_w])
        pending = fire_wave(w + 1) if w + 1 < n_waves else []

    lane = lax.iota(jnp.int32, L)
    for blk in range(BPW // L):
        rows16 = lane + blk * L
        acc_v = jnp.zeros((L,), jnp.float32)
        for d in range(D):
            cols = jnp.full((L,), d, jnp.int32)
            wv = plsc.load_gather(rows_w, [rows16, cols])
            cv = plsc.load_gather(rows_c, [rows16, cols])
            acc_v = acc_v + wv * cv
        acc[pl.ds(blk * L, L)] = acc_v

    pltpu.sync_copy(acc, out_hbm.at[pl.ds(base, BPW)])


def kernel(inputs, word_embeddings, context_embeddings):
    idx_c = inputs[:, 0].astype(jnp.int32)
    idx_w = inputs[:, 1].astype(jnp.int32)
    return _neg_sampling_dot(idx_c, idx_w,
                             context_embeddings.T, word_embeddings.T)


# speedup vs baseline: 9.8351x; 1.0354x over previous
import functools

import jax
import jax.numpy as jnp
from jax import lax
from jax.experimental import pallas as pl
from jax.experimental.pallas import tpu as pltpu
from jax.experimental.pallas import tpu_sc as plsc

B = 4096
D = 32
L = 16
NC = 2
NS = 16
NW = NC * NS
BPW = B // NW
WAVE = 8

_mesh = plsc.VectorSubcoreMesh(core_axis_name="c", subcore_axis_name="s")


@functools.partial(
    pl.kernel,
    mesh=_mesh,
    out_type=jax.ShapeDtypeStruct((B,), jnp.float32),
    scratch_types=[
        pltpu.VMEM((BPW,), jnp.int32),
        pltpu.VMEM((BPW,), jnp.int32),
        pltpu.VMEM((WAVE, D, 128), jnp.float32),
        pltpu.VMEM((WAVE, D, 128), jnp.float32),
        pltpu.VMEM((BPW, D), jnp.float32),
        pltpu.VMEM((BPW, D), jnp.float32),
        pltpu.VMEM((BPW,), jnp.float32),
        pltpu.SemaphoreType.DMA,
    ],
    compiler_params=pltpu.CompilerParams(needs_layout_passes=False),
)
def _neg_sampling_dot(inputs_t_hbm, ctx_t_hbm, word_t_hbm, out_hbm,
                      idx_c_v, idx_w_v, blk_c, blk_w, rows_c, rows_w,
                      acc, sem):
    wid = lax.axis_index("s") * NC + lax.axis_index("c")
    base = wid * BPW

    pltpu.sync_copy(inputs_t_hbm.at[0, pl.ds(base, BPW)], idx_c_v)
    pltpu.sync_copy(inputs_t_hbm.at[1, pl.ds(base, BPW)], idx_w_v)

    sc_c, sc_w = [], []
    for blk in range(BPW // L):
        vcs = idx_c_v[pl.ds(blk * L, L)]
        vws = idx_w_v[pl.ds(blk * L, L)]
        for r in range(L):
            sc_c.append(lax.index_in_dim(vcs, r, keepdims=False))
            sc_w.append(lax.index_in_dim(vws, r, keepdims=False))

    d_lo = lax.iota(jnp.int32, L)
    d_hi = d_lo + L
    lane = lax.iota(jnp.int32, L)

    def fire_wave(w):
        cps = []
        for r in range(WAVE):
            j = w * WAVE + r
            off_c = pl.multiple_of((sc_c[j] >> 7) * 128, 128)
            off_w = pl.multiple_of((sc_w[j] >> 7) * 128, 128)
            cps.append(pltpu.async_copy(
                ctx_t_hbm.at[:, pl.ds(off_c, 128)], blk_c.at[r], sem))
            cps.append(pltpu.async_copy(
                word_t_hbm.at[:, pl.ds(off_w, 128)], blk_w.at[r], sem))
        return cps

    def dot_block(b16):
        rows16 = lane + b16 * L
        acc_v = jnp.zeros((L,), jnp.float32)
        for d in range(D):
            cols = jnp.full((L,), d, jnp.int32)
            wv = plsc.load_gather(rows_w, [rows16, cols])
            cv = plsc.load_gather(rows_c, [rows16, cols])
            acc_v = acc_v + wv * cv
        acc[pl.ds(b16 * L, L)] = acc_v

    n_waves = BPW // WAVE
    pending = fire_wave(0)
    for w in range(n_waves):
        for cp in pending:
            cp.wait()
        for r in range(WAVE):
            j = w * WAVE + r
            vr_c = jnp.full((L,), sc_c[j] & 127, jnp.int32)
            vr_w = jnp.full((L,), sc_w[j] & 127, jnp.int32)
            rows_c[j, pl.ds(0, L)] = plsc.load_gather(blk_c.at[r], [d_lo, vr_c])
            rows_c[j, pl.ds(L, L)] = plsc.load_gather(blk_c.at[r], [d_hi, vr_c])
            rows_w[j, pl.ds(0, L)] = plsc.load_gather(blk_w.at[r], [d_lo, vr_w])
            rows_w[j, pl.ds(L, L)] = plsc.load_gather(blk_w.at[r], [d_hi, vr_w])
        pending = fire_wave(w + 1) if w + 1 < n_waves else []
        if (w + 1) * WAVE % L == 0:
            dot_block(((w + 1) * WAVE) // L - 1)

    pltpu.sync_copy(acc, out_hbm.at[pl.ds(base, BPW)])


def kernel(inputs, word_embeddings, context_embeddings):
    return _neg_sampling_dot(inputs.T.astype(jnp.int32),
                             context_embeddings.T, word_embeddings.T)


# speedup vs baseline: 9.8692x; 1.0035x over previous
import functools

import jax
import jax.numpy as jnp
from jax import lax
from jax.experimental import pallas as pl
from jax.experimental.pallas import tpu as pltpu
from jax.experimental.pallas import tpu_sc as plsc

B = 4096
D = 32
L = 16
NC = 2
NS = 16
NW = NC * NS
BPW = B // NW
WAVE = 8

_mesh = plsc.VectorSubcoreMesh(core_axis_name="c", subcore_axis_name="s")


@functools.partial(
    pl.kernel,
    mesh=_mesh,
    out_type=jax.ShapeDtypeStruct((B,), jnp.float32),
    scratch_types=[
        pltpu.VMEM((BPW,), jnp.int32),
        pltpu.VMEM((BPW,), jnp.int32),
        pltpu.VMEM((WAVE, D, 128), jnp.float32),
        pltpu.VMEM((WAVE, D, 128), jnp.float32),
        pltpu.VMEM((BPW, D), jnp.float32),
        pltpu.VMEM((BPW, D), jnp.float32),
        pltpu.VMEM((BPW,), jnp.float32),
        pltpu.SemaphoreType.DMA,
    ],
    compiler_params=pltpu.CompilerParams(
        needs_layout_passes=False,
        skip_device_barrier=True,
        disable_bounds_checks=True,
        disable_semaphore_checks=True,
    ),
)
def _neg_sampling_dot(inputs_t_hbm, ctx_t_hbm, word_t_hbm, out_hbm,
                      idx_c_v, idx_w_v, blk_c, blk_w, rows_c, rows_w,
                      acc, sem):
    wid = lax.axis_index("s") * NC + lax.axis_index("c")
    base = wid * BPW

    pltpu.sync_copy(inputs_t_hbm.at[0, pl.ds(base, BPW)], idx_c_v)
    pltpu.sync_copy(inputs_t_hbm.at[1, pl.ds(base, BPW)], idx_w_v)

    sc_c, sc_w = [], []
    for blk in range(BPW // L):
        vcs = idx_c_v[pl.ds(blk * L, L)]
        vws = idx_w_v[pl.ds(blk * L, L)]
        for r in range(L):
            sc_c.append(lax.index_in_dim(vcs, r, keepdims=False))
            sc_w.append(lax.index_in_dim(vws, r, keepdims=False))

    d_lo = lax.iota(jnp.int32, L)
    d_hi = d_lo + L
    lane = lax.iota(jnp.int32, L)

    def fire_wave(w):
        cps = []
        for r in range(WAVE):
            j = w * WAVE + r
            off_c = pl.multiple_of((sc_c[j] >> 7) * 128, 128)
            off_w = pl.multiple_of((sc_w[j] >> 7) * 128, 128)
            cps.append(pltpu.async_copy(
                ctx_t_hbm.at[:, pl.ds(off_c, 128)], blk_c.at[r], sem))
            cps.append(pltpu.async_copy(
                word_t_hbm.at[:, pl.ds(off_w, 128)], blk_w.at[r], sem))
        return cps

    def dot_block(b16):
        rows16 = lane + b16 * L
        acc_v = jnp.zeros((L,), jnp.float32)
        for d in range(D):
            cols = jnp.full((L,), d, jnp.int32)
            wv = plsc.load_gather(rows_w, [rows16, cols])
            cv = plsc.load_gather(rows_c, [rows16, cols])
            acc_v = acc_v + wv * cv
        acc[pl.ds(b16 * L, L)] = acc_v

    n_waves = BPW // WAVE
    pending = fire_wave(0)
    for w in range(n_waves):
        for cp in pending:
            cp.wait()
        for r in range(WAVE):
            j = w * WAVE + r
            vr_c = jnp.full((L,), sc_c[j] & 127, jnp.int32)
            vr_w = jnp.full((L,), sc_w[j] & 127, jnp.int32)
            rows_c[j, pl.ds(0, L)] = plsc.load_gather(blk_c.at[r], [d_lo, vr_c])
            rows_c[j, pl.ds(L, L)] = plsc.load_gather(blk_c.at[r], [d_hi, vr_c])
            rows_w[j, pl.ds(0, L)] = plsc.load_gather(blk_w.at[r], [d_lo, vr_w])
            rows_w[j, pl.ds(L, L)] = plsc.load_gather(blk_w.at[r], [d_hi, vr_w])
        pending = fire_wave(w + 1) if w + 1 < n_waves else []
        if (w + 1) * WAVE % L == 0:
            dot_block(((w + 1) * WAVE) // L - 1)

    pltpu.sync_copy(acc, out_hbm.at[pl.ds(base, BPW)])


def kernel(inputs, word_embeddings, context_embeddings):
    return _neg_sampling_dot(inputs.T.astype(jnp.int32),
                             context_embeddings.T, word_embeddings.T)


# speedup vs baseline: 10.3277x; 1.0465x over previous
import functools

import jax
import jax.numpy as jnp
from jax import lax
from jax.experimental import pallas as pl
from jax.experimental.pallas import tpu as pltpu
from jax.experimental.pallas import tpu_sc as plsc

B = 4096
D = 32
L = 16
NC = 2
NS = 16
NW = NC * NS
BPW = B // NW
WAVE = 4
DEPTH = 2

_mesh = plsc.VectorSubcoreMesh(core_axis_name="c", subcore_axis_name="s")


@functools.partial(
    pl.kernel,
    mesh=_mesh,
    out_type=jax.ShapeDtypeStruct((B,), jnp.float32),
    scratch_types=[
        pltpu.VMEM((BPW,), jnp.int32),
        pltpu.VMEM((BPW,), jnp.int32),
        pltpu.VMEM((DEPTH * WAVE, D, 128), jnp.float32),
        pltpu.VMEM((DEPTH * WAVE, D, 128), jnp.float32),
        pltpu.VMEM((BPW, D), jnp.float32),
        pltpu.VMEM((BPW, D), jnp.float32),
        pltpu.VMEM((BPW,), jnp.float32),
        pltpu.SemaphoreType.DMA,
        pltpu.SemaphoreType.DMA,
    ],
    compiler_params=pltpu.CompilerParams(needs_layout_passes=False),
)
def _neg_sampling_dot(inputs_t_hbm, ctx_t_hbm, word_t_hbm, out_hbm,
                      idx_c_v, idx_w_v, blk_c, blk_w, rows_c, rows_w,
                      acc, sem_a, sem_b):
    wid = lax.axis_index("s") * NC + lax.axis_index("c")
    base = wid * BPW

    pltpu.sync_copy(inputs_t_hbm.at[0, pl.ds(base, BPW)], idx_c_v)
    pltpu.sync_copy(inputs_t_hbm.at[1, pl.ds(base, BPW)], idx_w_v)

    sc_c, sc_w = [], []
    for blk in range(BPW // L):
        vcs = idx_c_v[pl.ds(blk * L, L)]
        vws = idx_w_v[pl.ds(blk * L, L)]
        for r in range(L):
            sc_c.append(lax.index_in_dim(vcs, r, keepdims=False))
            sc_w.append(lax.index_in_dim(vws, r, keepdims=False))

    d_lo = lax.iota(jnp.int32, L)
    d_hi = d_lo + L
    lane = lax.iota(jnp.int32, L)

    sems = (sem_a, sem_b)

    def fire_wave(w):
        slot = (w % DEPTH) * WAVE
        sem = sems[w % DEPTH]
        cps = []
        for r in range(WAVE):
            j = w * WAVE + r
            off_c = pl.multiple_of((sc_c[j] >> 7) * 128, 128)
            off_w = pl.multiple_of((sc_w[j] >> 7) * 128, 128)
            cps.append(pltpu.async_copy(
                ctx_t_hbm.at[:, pl.ds(off_c, 128)], blk_c.at[slot + r], sem))
            cps.append(pltpu.async_copy(
                word_t_hbm.at[:, pl.ds(off_w, 128)], blk_w.at[slot + r], sem))
        return cps

    def dot_block(b16):
        rows16 = lane + b16 * L
        acc_v = jnp.zeros((L,), jnp.float32)
        for d in range(D):
            cols = jnp.full((L,), d, jnp.int32)
            wv = plsc.load_gather(rows_w, [rows16, cols])
            cv = plsc.load_gather(rows_c, [rows16, cols])
            acc_v = acc_v + wv * cv
        acc[pl.ds(b16 * L, L)] = acc_v

    n_waves = BPW // WAVE
    inflight = [fire_wave(w) for w in range(DEPTH)]
    for w in range(n_waves):
        for cp in inflight[w % DEPTH]:
            cp.wait()
        slot = (w % DEPTH) * WAVE
        for r in range(WAVE):
            j = w * WAVE + r
            vr_c = jnp.full((L,), sc_c[j] & 127, jnp.int32)
            vr_w = jnp.full((L,), sc_w[j] & 127, jnp.int32)
            b_c = blk_c.at[slot + r]
            b_w = blk_w.at[slot + r]
            rows_c[j, pl.ds(0, L)] = plsc.load_gather(b_c, [d_lo, vr_c])
            rows_c[j, pl.ds(L, L)] = plsc.load_gather(b_c, [d_hi, vr_c])
            rows_w[j, pl.ds(0, L)] = plsc.load_gather(b_w, [d_lo, vr_w])
            rows_w[j, pl.ds(L, L)] = plsc.load_gather(b_w, [d_hi, vr_w])
        if w + DEPTH < n_waves:
            inflight[w % DEPTH] = fire_wave(w + DEPTH)
        if (w + 1) * WAVE % L == 0:
            dot_block(((w + 1) * WAVE) // L - 1)

    pltpu.sync_copy(acc, out_hbm.at[pl.ds(base, BPW)])


def kernel(inputs, word_embeddings, context_embeddings):
    return _neg_sampling_dot(inputs.T.astype(jnp.int32),
                             context_embeddings.T, word_embeddings.T)
